# baseline (speedup 1.0000x reference)
import functools

import jax
import jax.numpy as jnp
from jax import lax
from jax.experimental import pallas as pl
from jax.experimental.pallas import tpu as pltpu
from jax.experimental.pallas import tpu_sc as plsc

NUM_CORES = 2
NUM_SUBCORES = 16
NUM_WORKERS = NUM_CORES * NUM_SUBCORES

LANES = 128
RING = 3


def _make_lookup(seq, batch, dim):
  nb = batch // LANES
  n_slabs = seq * nb
  assert n_slabs % NUM_WORKERS == 0
  per_w = n_slabs // NUM_WORKERS
  na = dim // 8
  mesh = plsc.VectorSubcoreMesh(core_axis_name="c", subcore_axis_name="s")

  @functools.partial(
      pl.kernel,
      out_type=jax.ShapeDtypeStruct((seq * batch * dim // LANES, LANES),
                                    jnp.float32),
      mesh=mesh,
      scratch_types=[
          pltpu.VMEM((RING, LANES), jnp.int32),
          pltpu.VMEM((RING, LANES), jnp.int32),
          pltpu.VMEM((RING, LANES), jnp.int32),
          pltpu.VMEM((RING, LANES, LANES), jnp.float32),
          pltpu.VMEM((2, na, 8, LANES), jnp.float32),
          pltpu.SemaphoreType.DMA((RING,)),
          pltpu.SemaphoreType.DMA((RING,)),
          pltpu.SemaphoreType.DMA((2,)),
      ],
      compiler_params=pltpu.CompilerParams(
          use_tc_tiling_on_sc=False, needs_layout_passes=False),
  )
  def lookup(idxt_hbm, table2_hbm, out_hbm, idx_v, pair_v, par_v, buf_v,
             slab_v, idx_sem, gat_sem, out_sem):
    wid = lax.axis_index("s") * NUM_CORES + lax.axis_index("c")
    s0 = wid * per_w
    iota = lax.iota(jnp.int32, 16)

    def idx_load(slot, s):
      i1 = s // nb
      col = (s % nb) * LANES
      return pltpu.make_async_copy(
          idxt_hbm.at[i1, pl.ds(col, LANES)],
          idx_v.at[slot],
          idx_sem.at[slot],
      )

    def gather(slot):
      return pltpu.make_async_copy(
          table2_hbm.at[pair_v.at[slot]],
          buf_v.at[slot],
          gat_sem.at[slot],
      )

    def prep(slot):
      for g in range(LANES // 16):
        sl = pl.ds(g * 16, 16)
        raw = idx_v[slot, sl]
        pair_v[slot, sl] = lax.shift_right_logical(raw, 1)
        par_v[slot, sl] = lax.shift_left(lax.bitwise_and(raw, 1), 6)

    def out_store(p, s, a):
      i1 = s // nb
      b = s % nb
      row = i1 * (nb * dim) + a * (nb * 8) + b * 8
      return pltpu.make_async_copy(
          slab_v.at[p, a],
          out_hbm.at[pl.ds(row, 8)],
          out_sem.at[p],
      )

    def select(slot, p):
      for g in range(LANES // 16):
        rows = g * 16 + iota
        cols = par_v[slot, pl.ds(g * 16, 16)]
        for j in range(dim):
          v = plsc.load_gather(buf_v.at[slot], [rows, cols + j])
          slab_v[p, j // 8, j % 8, pl.ds(g * 16, 16)] = v

    for r in range(RING):
      idx_load(r, s0 + r).start()

    def body(t, _):
      s = s0 + t
      slot = lax.rem(t, RING)
      p = lax.rem(t, 2)
      idx_load(slot, s).wait()
      prep(slot)
      gather(slot).start()

      @pl.when(t >= 1)
      def _():
        pslot = lax.rem(t - 1, RING)
        pp = lax.rem(t - 1, 2)
        @pl.when(t >= 3)
        def _():
          for a in range(na):
            out_store(pp, s - 3, a).wait()
        gather(pslot).wait()
        select(pslot, pp)
        for a in range(na):
          out_store(pp, s - 1, a).start()

      @pl.when(t + RING < per_w)
      def _():
        idx_load(slot, s + RING).start()

      return ()

    lax.fori_loop(0, per_w, body, (), unroll=False)

    t_last = per_w - 1
    lslot = lax.rem(t_last, RING)
    lp = lax.rem(t_last, 2)
    @pl.when(per_w >= 3)
    def _():
      for a in range(na):
        out_store(lp, s0 + t_last - 2, a).wait()
    gather(lslot).wait()
    select(lslot, lp)
    for a in range(na):
      out_store(lp, s0 + t_last, a).start()
    for a in range(na):
      out_store(lp, s0 + t_last, a).wait()
    @pl.when(per_w >= 2)
    def _():
      pp = lax.rem(t_last - 1, 2)
      for a in range(na):
        out_store(pp, s0 + t_last - 1, a).wait()

  return lookup


def kernel(indices, table):
  b, s = indices.shape
  vocab, dim = table.shape
  idxt = indices.T.astype(jnp.int32)
  table2 = table.reshape(vocab * dim // 128, 128)
  out2 = _make_lookup(s, b, dim)(idxt, table2)
  p5 = out2.reshape(s, dim // 8, b // 128, 8, 128)
  return p5.transpose(2, 4, 0, 1, 3).reshape(b, s, dim)


# speedup vs baseline: 1.6754x; 1.6754x over previous
import functools

import jax
import jax.numpy as jnp
from jax import lax
from jax.experimental import pallas as pl
from jax.experimental.pallas import tpu as pltpu
from jax.experimental.pallas import tpu_sc as plsc

NUM_CORES = 2
NUM_SUBCORES = 16
NUM_WORKERS = NUM_CORES * NUM_SUBCORES

PBLK = 1024
CHUNK = 256
NBUF = 3
D_I0B = 128


def _pair_block(x1_ref, x2_ref, out_ref):
  out_ref[...] = jnp.concatenate([x1_ref[...].T, x2_ref[...].T], axis=1)


def _build_pair_table(table_t):
  dim, vocab = table_t.shape
  grid = pl.cdiv(vocab, 2 * PBLK)
  _last_blk = pl.cdiv(vocab, PBLK) - 1
  return pl.pallas_call(
      _pair_block,
      out_shape=jax.ShapeDtypeStruct((grid * PBLK, 2 * dim), jnp.float32),
      grid=(grid,),
      in_specs=[
          pl.BlockSpec((dim, PBLK),
                       lambda i: (0, jnp.minimum(2 * i, _last_blk))),
          pl.BlockSpec((dim, PBLK),
                       lambda i: (0, jnp.minimum(2 * i + 1, _last_blk))),
      ],
      out_specs=pl.BlockSpec((PBLK, 2 * dim), lambda i: (i, 0)),
  )(table_t, table_t)


def _make_lookup(total, dim, tab_rows):
  assert total % (NUM_WORKERS * CHUNK) == 0
  b_per_w = total // NUM_WORKERS
  n_chunks = b_per_w // CHUNK
  assert n_chunks >= NBUF
  mesh = plsc.VectorSubcoreMesh(core_axis_name="c", subcore_axis_name="s")

  @functools.partial(
      pl.kernel,
      out_type=jax.ShapeDtypeStruct((total, 2 * dim), jnp.float32),
      mesh=mesh,
      scratch_types=[
          pltpu.VMEM((NBUF, CHUNK), jnp.int32),
          pltpu.VMEM((NBUF, CHUNK, 2 * dim), jnp.float32),
          pltpu.SemaphoreType.DMA((NBUF,)),
          pltpu.SemaphoreType.DMA((NBUF,)),
          pltpu.SemaphoreType.DMA((NBUF,)),
      ],
      compiler_params=pltpu.CompilerParams(
          use_tc_tiling_on_sc=False, needs_layout_passes=False),
  )
  def lookup(idx_hbm, tab_hbm, out_hbm, idx_v, rows_v, idx_sem, row_sem,
             out_sem):
    wid = lax.axis_index("s") * NUM_CORES + lax.axis_index("c")
    base = wid * b_per_w

    def idx_load(slot, chunk):
      return pltpu.make_async_copy(
          idx_hbm.at[pl.ds(base + chunk * CHUNK, CHUNK)],
          idx_v.at[slot],
          idx_sem.at[slot],
      )

    def gather(slot):
      return pltpu.make_async_copy(
          tab_hbm.at[idx_v.at[slot]],
          rows_v.at[slot],
          row_sem.at[slot],
      )

    def store(slot, chunk):
      return pltpu.make_async_copy(
          rows_v.at[slot],
          out_hbm.at[pl.ds(base + chunk * CHUNK, CHUNK)],
          out_sem.at[slot],
      )

    for r in range(NBUF):
      idx_load(r, r).start()

    def body(c, _):
      slot = lax.rem(c, NBUF)
      idx_load(slot, c).wait()
      @pl.when(c >= NBUF)
      def _():
        store(slot, c - NBUF).wait()

      gather(slot).start()
      gather(slot).wait()
      store(slot, c).start()
      @pl.when(c + NBUF < n_chunks)
      def _():
        idx_load(slot, c + NBUF).start()

      return ()

    lax.fori_loop(0, n_chunks, body, (), unroll=False)

    for k in range(NBUF):
      c = n_chunks - NBUF + k
      store(c % NBUF, c).wait()

  return lookup


def _select_block(pairs_ref, idxt_ref, out_ref, *, seq, dim):
  half = (idxt_ref[...] >> 10) & 1
  x = pairs_ref[...].reshape(D_I0B, seq, 2 * dim)
  xt = jnp.transpose(x, (1, 2, 0))
  sel = jnp.where((half == 1)[:, None, :], xt[:, dim:, :], xt[:, :dim, :])
  out_ref[...] = sel.reshape(seq, dim // 8, 1, 8, D_I0B)


def _select_transpose(pairs, idxt, batch, seq, dim):
  grid = batch // D_I0B
  return pl.pallas_call(
      functools.partial(_select_block, seq=seq, dim=dim),
      out_shape=jax.ShapeDtypeStruct((seq, dim // 8, grid, 8, D_I0B),
                                     jnp.float32),
      grid=(grid,),
      in_specs=[
          pl.BlockSpec((D_I0B * seq, 2 * dim), lambda i: (i, 0)),
          pl.BlockSpec((seq, D_I0B), lambda i: (0, i)),
      ],
      out_specs=pl.BlockSpec((seq, dim // 8, 1, 8, D_I0B),
                             lambda i: (0, 0, i, 0, 0)),
  )(pairs, idxt)


def kernel(indices, table):
  b, s = indices.shape
  total = b * s
  vocab, dim = table.shape
  idx = indices.reshape(total).astype(jnp.int32)
  idx2 = ((idx >> 11) << 10) | (idx & (PBLK - 1))
  tab2 = _build_pair_table(table.T)
  pairs = _make_lookup(total, dim, tab2.shape[0])(idx2, tab2)
  idxt = indices.T.astype(jnp.int32)
  out5 = _select_transpose(pairs, idxt, b, s, dim)
  return out5.transpose(2, 4, 0, 1, 3).reshape(b, s, dim)
